# Initial kernel scaffold; baseline (speedup 1.0000x reference)
import jax
import jax.numpy as jnp
from jax.experimental import pallas as pl


def kernel(inputs, conds, w):
    raise NotImplementedError("write your pallas kernel here")


# speedup vs baseline: 3.6261x; 3.6261x over previous
import functools

import jax
import jax.numpy as jnp
from jax import lax
from jax.experimental import pallas as pl
from jax.experimental.pallas import tpu as pltpu
from jax.experimental.pallas import tpu_sc as plsc

_N = 4096
_B = 16384
_LSE_BLK = 256


def _lse_body(w_ref, lse_ref):
    x = w_ref[...]
    m = jnp.max(x, axis=1, keepdims=True)
    s = jnp.sum(jnp.exp(x - m), axis=1, keepdims=True)
    lse_ref[...] = (m + jnp.log(s))[:, 0]


def _row_logsumexp(w):
    return pl.pallas_call(
        _lse_body,
        grid=(_N // _LSE_BLK,),
        in_specs=[pl.BlockSpec((_LSE_BLK, _N), lambda i: (i, 0))],
        out_specs=pl.BlockSpec((_LSE_BLK,), lambda i: (i,)),
        out_shape=jax.ShapeDtypeStruct((_N,), jnp.float32),
    )(w)


@functools.cache
def _make_sc_gather():
    info = plsc.get_sparse_core_info()
    nc, ns, lanes = info.num_cores, info.num_subcores, info.num_lanes
    nw = nc * ns
    b_per_w = _B // nw
    ch = b_per_w // 128

    mesh = plsc.VectorSubcoreMesh(core_axis_name="c", subcore_axis_name="s")

    @functools.partial(
        pl.kernel,
        mesh=mesh,
        out_type=jax.ShapeDtypeStruct((_B,), jnp.float32),
        scratch_types=[
            pltpu.VMEM((ch, 128), jnp.int32),
            pltpu.VMEM((ch, 128), jnp.int32),
            pltpu.VMEM((ch, 128), jnp.int32),
            pltpu.VMEM((b_per_w,), jnp.float32),
            pltpu.VMEM((b_per_w,), jnp.float32),
            pltpu.VMEM((b_per_w,), jnp.float32),
            pltpu.SemaphoreType.DMA,
        ],
    )
    def sc_k(wflat_hbm, lse_hbm, conds_hbm, inputs_hbm, out_hbm,
             conds_v, inputs_v, fidx_v, vals_v, lseg_v, out_v, sem):
        wid = lax.axis_index("s") * nc + lax.axis_index("c")
        base = wid * b_per_w
        pltpu.sync_copy(conds_hbm.at[wid], conds_v)
        pltpu.sync_copy(inputs_hbm.at[wid], inputs_v)
        for i in range(ch):
            for j in range(128 // lanes):
                sl = pl.ds(j * lanes, lanes)
                fidx_v[i, sl] = conds_v[i, sl] * _N + inputs_v[i, sl]
        copies = []
        for i in range(ch):
            copies.append(pltpu.async_copy(
                wflat_hbm.at[fidx_v.at[i]],
                vals_v.at[pl.ds(i * 128, 128)], sem))
            copies.append(pltpu.async_copy(
                lse_hbm.at[conds_v.at[i]],
                lseg_v.at[pl.ds(i * 128, 128)], sem))
        for c in copies:
            c.wait()
        for i in range(b_per_w // lanes):
            sl = pl.ds(i * lanes, lanes)
            out_v[sl] = vals_v[sl] - lseg_v[sl]
        pltpu.sync_copy(out_v, out_hbm.at[pl.ds(base, b_per_w)])

    return sc_k, nw, ch


def kernel(inputs, conds, w):
    sc_k, nw, ch = _make_sc_gather()
    conds_3d = conds.reshape(nw, ch, 128).astype(jnp.int32)
    inputs_3d = inputs.reshape(nw, ch, 128).astype(jnp.int32)
    lse = _row_logsumexp(w)
    return sc_k(w.reshape(-1), lse, conds_3d, inputs_3d)


# speedup vs baseline: 5.0511x; 1.3930x over previous
import functools

import jax
import jax.numpy as jnp
from jax import lax
from jax.experimental import pallas as pl
from jax.experimental.pallas import tpu as pltpu
from jax.experimental.pallas import tpu_sc as plsc

_N = 4096
_B = 16384
_LSE_BLK = 256
_KC = _N // 128


def _z_body(w_ref, z_ref):
    x = w_ref[...]
    m = jnp.max(x, axis=1, keepdims=True)
    s = jnp.sum(jnp.exp(x - m), axis=1, keepdims=True)
    lse = m + jnp.log(s)
    for k in range(_KC):
        z_ref[k] = x[:, k * 128:(k + 1) * 128] - lse


def _normalized_logits(w):
    return pl.pallas_call(
        _z_body,
        grid=(_N // _LSE_BLK,),
        in_specs=[pl.BlockSpec((_LSE_BLK, _N), lambda i: (i, 0))],
        out_specs=pl.BlockSpec((_KC, _LSE_BLK, 128), lambda i: (0, i, 0)),
        out_shape=jax.ShapeDtypeStruct((_KC, _N, 128), jnp.float32),
    )(w)


@functools.cache
def _make_sc_gather():
    info = plsc.get_sparse_core_info()
    nc, ns, lanes = info.num_cores, info.num_subcores, info.num_lanes
    nw = nc * ns
    b_per_w = _B // nw
    ch = b_per_w // 128

    mesh = plsc.VectorSubcoreMesh(core_axis_name="c", subcore_axis_name="s")

    @functools.partial(
        pl.kernel,
        mesh=mesh,
        out_type=jax.ShapeDtypeStruct((_B,), jnp.float32),
        scratch_types=[
            pltpu.VMEM((ch, 128), jnp.int32),
            pltpu.VMEM((ch, 128), jnp.int32),
            pltpu.VMEM((ch, 128), jnp.int32),
            pltpu.VMEM((b_per_w,), jnp.float32),
            pltpu.SemaphoreType.DMA,
        ],
    )
    def sc_k(zlin_hbm, conds_hbm, inputs_hbm, out_hbm,
             conds_v, inputs_v, fidx_v, out_v, sem):
        wid = lax.axis_index("s") * nc + lax.axis_index("c")
        base = wid * b_per_w
        pltpu.sync_copy(conds_hbm.at[wid], conds_v)
        pltpu.sync_copy(inputs_hbm.at[wid], inputs_v)
        for i in range(ch):
            for j in range(128 // lanes):
                sl = pl.ds(j * lanes, lanes)
                inp = inputs_v[i, sl]
                fidx_v[i, sl] = ((inp >> 7) * _N + conds_v[i, sl]) * 128 \
                    + (inp & 127)
        copies = []
        for i in range(ch):
            copies.append(pltpu.async_copy(
                zlin_hbm.at[fidx_v.at[i]],
                out_v.at[pl.ds(i * 128, 128)], sem))
        for c in copies:
            c.wait()
        pltpu.sync_copy(out_v, out_hbm.at[pl.ds(base, b_per_w)])

    return sc_k, nw, ch


def kernel(inputs, conds, w):
    sc_k, nw, ch = _make_sc_gather()
    conds_3d = conds.reshape(nw, ch, 128).astype(jnp.int32)
    inputs_3d = inputs.reshape(nw, ch, 128).astype(jnp.int32)
    z3 = _normalized_logits(w)
    zlin = z3.reshape(_KC * _N * 128)
    return sc_k(zlin, conds_3d, inputs_3d)


# speedup vs baseline: 5.0883x; 1.0074x over previous
import functools

import jax
import jax.numpy as jnp
from jax import lax
from jax.experimental import pallas as pl
from jax.experimental.pallas import tpu as pltpu
from jax.experimental.pallas import tpu_sc as plsc

_N = 4096
_B = 16384
_LSE_BLK = 512
_KC = _N // 128


def _z_body(w_ref, z_ref):
    x = w_ref[...]
    m = jnp.max(x, axis=1, keepdims=True)
    s = jnp.sum(jnp.exp(x - m), axis=1, keepdims=True)
    lse = m + jnp.log(s)
    for k in range(_KC):
        z_ref[k] = x[:, k * 128:(k + 1) * 128] - lse


def _normalized_logits(w):
    return pl.pallas_call(
        _z_body,
        grid=(_N // _LSE_BLK,),
        in_specs=[pl.BlockSpec((_LSE_BLK, _N), lambda i: (i, 0))],
        out_specs=pl.BlockSpec((_KC, _LSE_BLK, 128), lambda i: (0, i, 0)),
        out_shape=jax.ShapeDtypeStruct((_KC, _N, 128), jnp.float32),
    )(w)


@functools.cache
def _make_sc_gather():
    info = plsc.get_sparse_core_info()
    nc, ns, lanes = info.num_cores, info.num_subcores, info.num_lanes
    nw = nc * ns
    b_per_w = _B // nw
    ch = b_per_w // 128

    mesh = plsc.VectorSubcoreMesh(core_axis_name="c", subcore_axis_name="s")

    @functools.partial(
        pl.kernel,
        mesh=mesh,
        out_type=jax.ShapeDtypeStruct((_B,), jnp.float32),
        scratch_types=[
            pltpu.VMEM((ch, 128), jnp.int32),
            pltpu.VMEM((ch, 128), jnp.int32),
            pltpu.VMEM((ch, 128), jnp.int32),
            pltpu.VMEM((b_per_w,), jnp.float32),
            pltpu.SemaphoreType.DMA,
        ],
    )
    def sc_k(zlin_hbm, conds_hbm, inputs_hbm, out_hbm,
             conds_v, inputs_v, fidx_v, out_v, sem):
        wid = lax.axis_index("s") * nc + lax.axis_index("c")
        base = wid * b_per_w
        pltpu.sync_copy(conds_hbm.at[wid], conds_v)
        pltpu.sync_copy(inputs_hbm.at[wid], inputs_v)
        for i in range(ch):
            for j in range(128 // lanes):
                sl = pl.ds(j * lanes, lanes)
                inp = inputs_v[i, sl]
                fidx_v[i, sl] = ((inp >> 7) * _N + conds_v[i, sl]) * 128 \
                    + (inp & 127)
        copies = []
        for i in range(ch):
            copies.append(pltpu.async_copy(
                zlin_hbm.at[fidx_v.at[i]],
                out_v.at[pl.ds(i * 128, 128)], sem))
        for c in copies:
            c.wait()
        pltpu.sync_copy(out_v, out_hbm.at[pl.ds(base, b_per_w)])

    return sc_k, nw, ch


def kernel(inputs, conds, w):
    sc_k, nw, ch = _make_sc_gather()
    conds_3d = conds.reshape(nw, ch, 128).astype(jnp.int32)
    inputs_3d = inputs.reshape(nw, ch, 128).astype(jnp.int32)
    z3 = _normalized_logits(w)
    zlin = z3.reshape(_KC * _N * 128)
    return sc_k(zlin, conds_3d, inputs_3d)
